# baseline (speedup 1.0000x reference)
import functools

import jax
import jax.numpy as jnp
from jax import lax
from jax.experimental import pallas as pl
from jax.experimental.pallas import tpu as pltpu
from jax.experimental.pallas import tpu_sc as plsc

H, W = 360, 480
NPIX = H * W
N = 100000
NP = 102400
TPTS = NP // 16
CH = 640
NCHUNK = TPTS // CH
MDC = CH * 7
MEC = CH * 2
PIX_C = NPIX // 16
ZLEN = 2160
LN2 = 0.6931471805599453
P5, P4, P3, P2, P1, P0 = (0.030102625011658456, -0.2806325404494927,
                          1.1048082361987304, -2.4208125632180866,
                          3.4982279012091095, -1.9316715417207186)


@functools.cache
def _make_sc():
  return pl.kernel(
    _sc_body,
    mesh=plsc.VectorSubcoreMesh(core_axis_name="c", subcore_axis_name="s",
                                num_cores=2, num_subcores=16),
    out_type=[jax.ShapeDtypeStruct((NPIX,), jnp.float32) for _ in range(5)],
    scratch_types=[
        [pltpu.VMEM((MDC,), jnp.float32) for _ in range(2)],
        [pltpu.VMEM((MEC,), jnp.float32) for _ in range(2)],
        [pltpu.VMEM((CH,), jnp.int32) for _ in range(2)],
        [[pltpu.VMEM((CH,), jnp.float32) for _ in range(3)]
         for _ in range(2)],
        pltpu.VMEM((ZLEN,), jnp.float32),
        [pltpu.VMEM_SHARED((NPIX,), jnp.float32) for _ in range(5)],
        pltpu.SemaphoreType.DMA,
        pltpu.SemaphoreType.DMA,
    ],
    compiler_params=pltpu.CompilerParams(needs_layout_passes=False),
  )


def _sc_body(md_hbm, me_hbm, o0, o1, o2, o3, o4,
             mdb, meb, idxb, valb, zbuf, accs, sem_s, sem_l):
    cc = lax.axis_index("c")
    ss = lax.axis_index("s")
    outs = (o0, o1, o2, o3, o4)
    scope = jax.named_scope

    lane = lax.iota(jnp.int32, 16)
    lane7 = lane * 7
    lane2 = lane * 2

    def _for_core(fn3, fn2):
        @pl.when(cc == 0)
        def _():
            fn3()
        @pl.when(cc != 0)
        def _():
            fn2()

    def _load(c, b):
        p0 = ss * TPTS + c * CH
        return [pltpu.async_copy(md_hbm.at[pl.ds(p0 * 7, MDC)], mdb[b],
                                 sem_l),
                pltpu.async_copy(me_hbm.at[pl.ds(p0 * 2, MEC)], meb[b],
                                 sem_l)]

    def _sigmoid(x):
        return 1.0 / (1.0 + jnp.exp(-x))

    def _softplus(x):
        ax = jnp.abs(x)
        y = 1.0 + jnp.exp(-ax)
        bits = plsc.bitcast(y, jnp.int32)
        m = plsc.bitcast(
            (bits & jnp.int32(0x007FFFFF)) | jnp.int32(0x3F800000),
            jnp.float32)
        ef = ((bits >> 23) - 127).astype(jnp.float32)
        lnm = ((((P5 * m + P4) * m + P3) * m + P2) * m + P1) * m + P0
        return jnp.maximum(x, 0.0) + LN2 * ef + lnm

    def _pix_idx(b, i, gid, valid):
        mx = plsc.load_gather(meb[b], [lane2 + i * 32])
        my = plsc.load_gather(meb[b], [lane2 + i * 32 + 1])
        px = jnp.clip((mx * W).astype(jnp.int32), 0, W - 1)
        py = jnp.clip((my * H).astype(jnp.int32), 0, H - 1)
        flat = py * W + px
        idxb[b][pl.ds(i * 16, 16)] = jnp.where(valid, flat, gid)

    def _compute(c, b):
        gbase = ss * TPTS + c * CH

        def _body3(i, carry):
            gid = gbase + i * 16 + lane
            valid = gid < N
            op = plsc.load_gather(mdb[b], [lane7 + i * 112 + 3])
            w = jnp.where(valid, _sigmoid(op), 0.0)
            for ch in range(3):
                cv = plsc.load_gather(mdb[b], [lane7 + i * 112 + ch])
                valb[b][ch][pl.ds(i * 16, 16)] = cv * w
            _pix_idx(b, i, gid, valid)
            return carry

        def _body2(i, carry):
            gid = gbase + i * 16 + lane
            valid = gid < N
            op = plsc.load_gather(mdb[b], [lane7 + i * 112 + 3])
            dp = plsc.load_gather(mdb[b], [lane7 + i * 112 + 4])
            w = jnp.where(valid, _sigmoid(op), 0.0)
            valb[b][0][pl.ds(i * 16, 16)] = _softplus(dp) * w
            valb[b][1][pl.ds(i * 16, 16)] = w
            _pix_idx(b, i, gid, valid)
            return carry
        _for_core(lambda: lax.fori_loop(0, CH // 16, _body3, 0),
                  lambda: lax.fori_loop(0, CH // 16, _body2, 0))

    pf0 = _load(0, 0)

    with scope("zero_phase"):
        def _zb(i, carry):
            zbuf[pl.ds(i * 16, 16)] = jnp.zeros((16,), jnp.float32)
            return carry
        lax.fori_loop(0, ZLEN // 16, _zb, 0)
        zcopies = []
        for a in accs:
            for k in range(PIX_C // ZLEN):
                zcopies.append(pltpu.async_copy(
                    zbuf, a.at[pl.ds(ss * PIX_C + k * ZLEN, ZLEN)], sem_l))
        for cp in zcopies:
            cp.wait()
        plsc.subcore_barrier()

    with scope("scatter_phase"):
        for cp in pf0:
            cp.wait()
        _compute(0, 0)
        for c in range(NCHUNK):
            b = c % 2
            nb = (c + 1) % 2

            def _scat(accl, nv, b=b):
                streams = [pltpu.async_copy(valb[b][k], accl[k].at[idxb[b]],
                                            sem_s, add=True)
                           for k in range(nv)]
                if c + 1 < NCHUNK:
                    for cp in _load(c + 1, nb):
                        cp.wait()
                    _compute(c + 1, nb)
                for cp in streams:
                    cp.wait()
            _for_core(lambda: _scat((accs[0], accs[1], accs[2]), 3),
                      lambda: _scat((accs[3], accs[4]), 2))
        plsc.subcore_barrier()

    with scope("copyout_phase"):
        def _cpout(chs):
            for ch in chs:
                for k, (off, ln) in enumerate(((0, 4480), (4480, 4480),
                                               (8960, 1840))):
                    bb = mdb[k % 2]
                    base = ss * PIX_C + off
                    pltpu.sync_copy(accs[ch].at[pl.ds(base, ln)],
                                    bb.at[pl.ds(0, ln)])
                    pltpu.sync_copy(bb.at[pl.ds(0, ln)],
                                    outs[ch].at[pl.ds(base, ln)])
        _for_core(lambda: _cpout((0, 1, 2)), lambda: _cpout((3, 4)))


def kernel(md_1_for, means2D):
    mdf = jnp.pad(md_1_for, ((0, NP - N), (0, 0))).reshape(NP * 7)
    mef = jnp.pad(means2D, ((0, NP - N), (0, 0))).reshape(NP * 2)
    c0a, c1a, c2a, dpa, wa = _make_sc()(mdf, mef)
    denom = jnp.maximum(wa, 1e-8)
    color = (jnp.stack([c0a, c1a, c2a], axis=-1)
             / denom[:, None]).reshape(H, W, 3)
    depth = (dpa / denom).reshape(H, W, 1)
    sil = jnp.clip(wa, 0.0, 1.0).reshape(H, W, 1)
    return (color, depth, sil)


# speedup vs baseline: 4.6626x; 4.6626x over previous
import functools

import jax
import jax.numpy as jnp
from jax import lax
from jax.experimental import pallas as pl
from jax.experimental.pallas import tpu as pltpu
from jax.experimental.pallas import tpu_sc as plsc

H, W = 360, 480
NPIX = H * W
N = 100000
NP = 102400
ROWS = NP // 128
TROWS = ROWS // 16
CB = 10
CPTS = CB * 128
NCHUNK = TROWS // CB
PIX_W = NPIX // 32
PIX_C = NPIX // 16
ZLEN = 2160


def _pre_body(c0, c1, c2, op, dp, mx, my, o0, o1, o2, o3, o4, idx_ref):
    w = 1.0 / (1.0 + jnp.exp(-op[...]))
    x = dp[...]
    d = jnp.maximum(x, 0.0) + jnp.log(1.0 + jnp.exp(-jnp.abs(x)))
    rid = lax.broadcasted_iota(jnp.int32, (ROWS, 128), 0)
    cid = lax.broadcasted_iota(jnp.int32, (ROWS, 128), 1)
    valid = (rid * 128 + cid) < N
    w = jnp.where(valid, w, 0.0)
    px = jnp.clip(jnp.floor(mx[...] * W), 0.0, W - 1)
    py = jnp.clip(jnp.floor(my[...] * H), 0.0, H - 1)
    flat = (py * W + px).astype(jnp.int32)
    idx_ref[...] = jnp.where(valid, flat, rid * 128 + cid)
    o0[...] = c0[...] * w
    o1[...] = c1[...] * w
    o2[...] = c2[...] * w
    o3[...] = d * w
    o4[...] = w


_pre = pl.pallas_call(
    _pre_body,
    out_shape=(
        [jax.ShapeDtypeStruct((ROWS, 128), jnp.float32) for _ in range(5)]
        + [jax.ShapeDtypeStruct((ROWS, 128), jnp.int32)]
    ),
)


@functools.cache
def _make_sc():
  return pl.kernel(
    _sc_body,
    mesh=plsc.VectorSubcoreMesh(core_axis_name="c", subcore_axis_name="s",
                                num_cores=2, num_subcores=16),
    out_type=[jax.ShapeDtypeStruct((NPIX,), jnp.float32) for _ in range(5)],
    scratch_types=[
        [pltpu.VMEM((CPTS,), jnp.int32) for _ in range(2)],
        [pltpu.VMEM((CPTS,), jnp.float32) for _ in range(5)],
        [pltpu.VMEM((CPTS,), jnp.float32) for _ in range(5)],
        pltpu.VMEM((PIX_W,), jnp.float32),
        [pltpu.VMEM_SHARED((NPIX,), jnp.float32) for _ in range(5)],
        pltpu.SemaphoreType.DMA,
        pltpu.SemaphoreType.DMA,
    ],
    compiler_params=pltpu.CompilerParams(needs_layout_passes=False),
  )


def _sc_body(v0h, v1h, v2h, v3h, v4h, idx_hbm, o0, o1, o2, o3, o4,
             idxv, vbufA, vbufB, cbuf, accs, sem_s, sem_l):
    cc = lax.axis_index("c")
    ss = lax.axis_index("s")
    wid = cc * 16 + ss
    vhbm = (v0h, v1h, v2h, v3h, v4h)
    outs = (o0, o1, o2, o3, o4)
    vbufs = (vbufA, vbufB)
    scope = jax.named_scope

    base_pt = ss * TROWS * 128

    def _for_core(fn3, fn2):
        @pl.when(cc == 0)
        def _():
            fn3()
        @pl.when(cc != 0)
        def _():
            fn2()

    def _load(c, buf, sem, chans):
        cps = [pltpu.async_copy(
            vhbm[ch].at[pl.ds(base_pt + c * CPTS, CPTS)], buf[ch], sem)
            for ch in chans]
        cps.append(pltpu.async_copy(
            idx_hbm.at[pl.ds(base_pt + c * CPTS, CPTS)], idxv[c % 2], sem))
        return cps

    pf3 = lambda: [None for _ in _load(0, vbufs[0], sem_s, (0, 1, 2))]
    pf2 = lambda: [None for _ in _load(0, vbufs[0], sem_s, (3, 4))]
    _for_core(pf3, pf2)

    with scope("zero_phase"):
        def _zb(i, carry):
            cbuf[pl.ds(i * 16, 16)] = jnp.zeros((16,), jnp.float32)
            return carry
        lax.fori_loop(0, PIX_W // 16, _zb, 0)
        zcopies = []
        for a in accs:
            for k in range(PIX_C // PIX_W):
                zcopies.append(pltpu.async_copy(
                    cbuf, a.at[pl.ds(ss * PIX_C + k * PIX_W, PIX_W)], sem_l))
        for cp in zcopies:
            cp.wait()
        plsc.subcore_barrier()

    with scope("scatter_phase"):
        def _drain_pf(nch):
            for _ in range(nch + 1):
                pltpu.make_async_copy(
                    vhbm[0].at[pl.ds(base_pt, CPTS)], vbufs[0][0],
                    sem_s).wait()
        _for_core(lambda: _drain_pf(3), lambda: _drain_pf(2))
        for c in range(NCHUNK):
            cur = vbufs[c % 2]

            def _scat(chs, cur=cur):
                streams = [pltpu.async_copy(cur[ch],
                                            accs[ch].at[idxv[c % 2]],
                                            sem_s, add=True) for ch in chs]
                if c + 1 < NCHUNK:
                    loads = _load(c + 1, vbufs[(c + 1) % 2], sem_l, chs)
                    for cp in loads:
                        cp.wait()
                for cp in streams:
                    cp.wait()
            _for_core(lambda: _scat((0, 1, 2)), lambda: _scat((3, 4)))
        plsc.subcore_barrier()

    with scope("copyout_phase"):
        def _cpout(chs):
            for ch in chs:
                for k in range(2):
                    base = ss * PIX_C + k * PIX_W
                    pltpu.sync_copy(accs[ch].at[pl.ds(base, PIX_W)], cbuf)
                    pltpu.sync_copy(cbuf, outs[ch].at[pl.ds(base, PIX_W)])
        _for_core(lambda: _cpout((0, 1, 2)), lambda: _cpout((3, 4)))


def kernel(md_1_for, means2D):
    mdp = jnp.pad(md_1_for, ((0, NP - N), (0, 0)))
    mep = jnp.pad(means2D, ((0, NP - N), (0, 0)))
    cols = [mdp[:, i].reshape(ROWS, 128) for i in (0, 1, 2, 3, 4)]
    mx = mep[:, 0].reshape(ROWS, 128)
    my = mep[:, 1].reshape(ROWS, 128)
    o0, o1, o2, o3, o4, idx = _pre(cols[0], cols[1], cols[2], cols[3],
                                   cols[4], mx, my)
    r1 = lambda a: a.reshape(NP)
    c0a, c1a, c2a, dpa, wa = _make_sc()(r1(o0), r1(o1), r1(o2), r1(o3),
                                        r1(o4), r1(idx))
    denom = jnp.maximum(wa, 1e-8)
    color = (jnp.stack([c0a, c1a, c2a], axis=-1)
             / denom[:, None]).reshape(H, W, 3)
    depth = (dpa / denom).reshape(H, W, 1)
    sil = jnp.clip(wa, 0.0, 1.0).reshape(H, W, 1)
    return (color, depth, sil)
